# Initial kernel scaffold; baseline (speedup 1.0000x reference)
import jax
import jax.numpy as jnp
from jax.experimental import pallas as pl


def kernel(Qa, Ra, batch_seg):
    raise NotImplementedError("write your pallas kernel here")


# speedup vs baseline: 2.1601x; 2.1601x over previous
import functools

import jax
import jax.numpy as jnp
from jax import lax
from jax.experimental import pallas as pl
from jax.experimental.pallas import tpu as pltpu
from jax.experimental.pallas import tpu_sc as plsc

N = 6_400_000
S = 100_000
SPAD = 100_352
W = 8
B = 5_120
NBLK = N // B
NCH = B // 128
NW = 32
ITERS = -(-NBLK // NW)
TSTRIPE = SPAD // 16
ZROWS = TSTRIPE // 4


def _sc_body(qa_hbm, ra_hbm, seg_hbm, zin_hbm, zpav_hbm, out_hbm,
             acc, qv, rv, segv, pav, zv, sem):
    cid = lax.axis_index("c")
    sid = lax.axis_index("s")
    wid = sid * 2 + cid
    tile = sid

    pltpu.sync_copy(zpav_hbm, pav)
    pltpu.sync_copy(zin_hbm, zv)
    for k in range(4):
        pltpu.sync_copy(zv, acc.at[pl.ds(tile * TSTRIPE + k * ZROWS, ZROWS)])
    plsc.subcore_barrier()

    iota = lax.iota(jnp.int32, 16)
    rcol = [iota * 3 + c for c in range(3)]

    def block_body(i, carry):
        b = i * NW + wid

        @pl.when(b < NBLK)
        def _():
            pltpu.sync_copy(qa_hbm.at[b], qv)
            pltpu.sync_copy(ra_hbm.at[b], rv)
            pltpu.sync_copy(seg_hbm.at[b], segv)

            def group(g, c2):
                a0 = g * 16
                q16 = qv[pl.ds(a0, 16)]
                avec = iota + a0
                base3 = a0 * 3
                for c in range(3):
                    rg = plsc.load_gather(rv, [rcol[c] + base3])
                    plsc.store_scatter(pav, [avec, jnp.full((16,), c, jnp.int32)],
                                       q16 * rg)
                return c2

            lax.fori_loop(0, B // 16, group, 0)

            handles = []
            for ch in range(NCH):
                handles.append(pltpu.async_copy(
                    pav.at[pl.ds(ch * 128, 128)],
                    acc.at[segv.at[ch]],
                    sem, add=True))
            for h in handles:
                h.wait()

        return carry

    lax.fori_loop(0, ITERS, block_body, 0)
    plsc.subcore_barrier()

    for k in range(4):
        r0 = tile * TSTRIPE + k * ZROWS
        pltpu.sync_copy(acc.at[pl.ds(r0, ZROWS)], zv)
        pltpu.sync_copy(zv, out_hbm.at[cid, pl.ds(r0, ZROWS)])


def _combine_body(a_ref, b_ref, o_ref):
    o_ref[...] = a_ref[...] + b_ref[...]


def kernel(Qa, Ra, batch_seg):
    seg32 = batch_seg.astype(jnp.int32)
    qa_blk = Qa.reshape(NBLK, B)
    ra_blk = Ra.reshape(NBLK, 3 * B)
    seg_blk = seg32.reshape(NBLK, NCH, 128)
    zin = jnp.zeros((ZROWS, W), jnp.float32)
    zpav = jnp.zeros((B, W), jnp.float32)

    mesh = plsc.VectorSubcoreMesh(core_axis_name="c", subcore_axis_name="s")
    sc = functools.partial(
        pl.kernel,
        mesh=mesh,
        out_type=jax.ShapeDtypeStruct((2, SPAD, W), jnp.float32),
        scratch_types=[
            pltpu.VMEM_SHARED((SPAD, W), jnp.float32),
            pltpu.VMEM((B,), jnp.float32),
            pltpu.VMEM((3 * B,), jnp.float32),
            pltpu.VMEM((NCH, 128), jnp.int32),
            pltpu.VMEM((B, W), jnp.float32),
            pltpu.VMEM((ZROWS, W), jnp.float32),
            pltpu.SemaphoreType.DMA,
        ],
        compiler_params=pltpu.CompilerParams(
            needs_layout_passes=False, use_tc_tiling_on_sc=False),
    )(_sc_body)
    partials = sc(qa_blk, ra_blk, seg_blk, zin, zpav)

    p2 = partials.reshape(2, SPAD * W // 128, 128)
    comb = pl.pallas_call(
        _combine_body,
        out_shape=jax.ShapeDtypeStruct((SPAD * W // 128, 128), jnp.float32),
    )(p2[0], p2[1])
    return comb.reshape(SPAD, W)[:S, :3]
